# baseline (speedup 1.0000x reference)
import jax
import jax.numpy as jnp
from jax.experimental import pallas as pl
from jax.experimental.pallas import tpu as pltpu
from jax.experimental.pallas import tpu_sc as plsc

_D = 64
_W = 64
_K = 8
_WB = _W * _K


def _gather_rows(pe, idx2d, n):
    mesh = plsc.VectorSubcoreMesh(core_axis_name="c", subcore_axis_name="s")

    @pl.kernel(
        out_type=jax.ShapeDtypeStruct((n, _D), pe.dtype),
        mesh=mesh,
        scratch_types=[pltpu.SemaphoreType.DMA],
        compiler_params=pltpu.CompilerParams(use_tc_tiling_on_sc=False),
    )
    def gather_kernel(pe_hbm, idx_hbm, out_hbm, sem):
        def body(idx_vmem, out_vmem):
            copies = [
                pltpu.async_copy(
                    pe_hbm.at[idx_vmem.at[0, pl.ds(j * _W, _W)]],
                    out_vmem.at[pl.ds(j * _W, _W)],
                    sem,
                )
                for j in range(_K)
            ]
            for c in copies:
                c.wait()

        pltpu.emit_pipeline(
            body,
            grid=(n // _WB,),
            in_specs=[pl.BlockSpec((1, _WB), index_map=lambda i: (0, i))],
            out_specs=[pl.BlockSpec((_WB, _D), index_map=lambda i: (i, 0))],
            core_axis_name=("c", "s"),
            dimension_semantics=(pltpu.PARALLEL,),
        )(idx_hbm, out_hbm)

    return gather_kernel(pe, idx2d)


def kernel(x, pe):
    b, t = x.shape
    n = b * t
    idx2d = x.reshape(1, n).astype(jnp.int32)
    out = _gather_rows(pe, idx2d, n)
    return out.reshape(b, t, _D)


# speedup vs baseline: 1.0959x; 1.0959x over previous
import jax
import jax.numpy as jnp
from jax.experimental import pallas as pl
from jax.experimental.pallas import tpu as pltpu
from jax.experimental.pallas import tpu_sc as plsc

_D = 64
_W = 64
_K = 2
_WB = _W * _K


def _gather_rows(pe, idx2d, n):
    mesh = plsc.VectorSubcoreMesh(core_axis_name="c", subcore_axis_name="s")

    @pl.kernel(
        out_type=jax.ShapeDtypeStruct((n, _D), pe.dtype),
        mesh=mesh,
        scratch_types=[pltpu.SemaphoreType.DMA],
        compiler_params=pltpu.CompilerParams(use_tc_tiling_on_sc=False),
    )
    def gather_kernel(pe_hbm, idx_hbm, out_hbm, sem):
        def body(idx_vmem, out_vmem):
            copies = [
                pltpu.async_copy(
                    pe_hbm.at[idx_vmem.at[0, pl.ds(j * _W, _W)]],
                    out_vmem.at[pl.ds(j * _W, _W)],
                    sem,
                )
                for j in range(_K)
            ]
            for c in copies:
                c.wait()

        pltpu.emit_pipeline(
            body,
            grid=(n // _WB,),
            in_specs=[pl.BlockSpec((1, _WB), index_map=lambda i: (0, i))],
            out_specs=[pl.BlockSpec((_WB, _D), index_map=lambda i: (i, 0))],
            core_axis_name=("c", "s"),
            dimension_semantics=(pltpu.PARALLEL,),
        )(idx_hbm, out_hbm)

    return gather_kernel(pe, idx2d)


def kernel(x, pe):
    b, t = x.shape
    n = b * t
    idx2d = x.reshape(1, n).astype(jnp.int32)
    out = _gather_rows(pe, idx2d, n)
    return out.reshape(b, t, _D)
